# Initial kernel scaffold; baseline (speedup 1.0000x reference)
import jax
import jax.numpy as jnp
from jax.experimental import pallas as pl


def kernel(x, edge_index, edge_type, edge_attr, W_rel1, W_root1, b1, W_rel2, W_root2, b2):
    raise NotImplementedError("write your pallas kernel here")


# speedup vs baseline: 3.4009x; 3.4009x over previous
import functools

import jax
import jax.numpy as jnp
from jax import lax
from jax.experimental import pallas as pl
from jax.experimental.pallas import tpu as pltpu
from jax.experimental.pallas import tpu_sc as plsc

_N = 10000
_E = 320000
_D = 128
_R = 8
_L = 16
_NCHUNK = _D // _L
_NC = 2
_NS = 16
_CPC = _NCHUNK // _NC
_RN = _R * _N
_B = 80
_EPT = _E // _NS
_EPW = _E // (_NC * _NS)
_SPT = _RN // _NS
_ZR = 1250

_mesh = plsc.VectorSubcoreMesh(core_axis_name="c", subcore_axis_name="s")
_sc_params = pltpu.CompilerParams(use_tc_tiling_on_sc=False)


def _fill(buf, nrows, value):
    def body(i, carry):
        buf[i, :] = jnp.full((_L,), value, jnp.float32)
        return carry
    lax.fori_loop(0, nrows, body, None)


def _clear_stripe(acc, zbuf, base):
    for q in range(_SPT // _ZR):
        pltpu.sync_copy(zbuf, acc.at[pl.ds(base + q * _ZR, _ZR)])


@functools.partial(
    pl.kernel,
    out_type=jax.ShapeDtypeStruct((_NC, _RN, _L), jnp.float32),
    mesh=_mesh,
    scratch_types=[
        pltpu.VMEM((_B,), jnp.int32),
        pltpu.VMEM((_B,), jnp.int32),
        pltpu.VMEM((_B,), jnp.int32),
        pltpu.VMEM((_B, _L), jnp.float32),
        pltpu.VMEM((_ZR, _L), jnp.float32),
        pltpu.VMEM_SHARED((_RN, _L), jnp.float32),
    ],
    compiler_params=_sc_params,
)
def _sc_count(dst_hbm, typ_hbm, out_hbm, dstb, typb, sidx, ones, zbuf, acc):
    c = lax.axis_index("c")
    s = lax.axis_index("s")
    _fill(ones, _B, 1.0)
    _fill(zbuf, _ZR, 0.0)
    base = s * _SPT
    _clear_stripe(acc, zbuf, base)
    plsc.subcore_barrier()

    estart = (c * _NS + s) * _EPW

    def batch(i, carry):
        eb = estart + i * _B
        pltpu.sync_copy(dst_hbm.at[pl.ds(eb, _B)], dstb)
        pltpu.sync_copy(typ_hbm.at[pl.ds(eb, _B)], typb)
        for j in range(_B // _L):
            sl = pl.ds(j * _L, _L)
            sidx[sl] = typb[sl] * _N + dstb[sl]
        pltpu.sync_copy(ones, acc.at[sidx], add=True)
        return carry

    lax.fori_loop(0, _EPW // _B, batch, None)
    plsc.subcore_barrier()
    pltpu.sync_copy(acc.at[pl.ds(base, _SPT)], out_hbm.at[c, pl.ds(base, _SPT)])


@functools.partial(
    pl.kernel,
    out_type=jax.ShapeDtypeStruct((_NCHUNK, _RN, _L), jnp.float32),
    mesh=_mesh,
    scratch_types=[
        pltpu.VMEM((_B,), jnp.int32),
        pltpu.VMEM((_B,), jnp.int32),
        pltpu.VMEM((_B,), jnp.int32),
        pltpu.VMEM((_B,), jnp.int32),
        pltpu.VMEM((_B,), jnp.int32),
        pltpu.VMEM((_B, _L), jnp.float32),
        pltpu.VMEM((_ZR, _L), jnp.float32),
        pltpu.VMEM_SHARED((_RN, _L), jnp.float32),
        pltpu.SemaphoreType.DMA,
    ],
    compiler_params=_sc_params,
)
def _sc_agg(xv_hbm, src_hbm, dst_hbm, typ_hbm, out_hbm,
            srcb, dstb, typb, gidx, sidx, rows, zbuf, acc, sem):
    c = lax.axis_index("c")
    s = lax.axis_index("s")
    _fill(zbuf, _ZR, 0.0)
    base = s * _SPT
    estart = s * _EPT

    for t in range(_CPC):
        k = c * _CPC + t
        _clear_stripe(acc, zbuf, base)
        plsc.subcore_barrier()

        def batch(i, carry):
            eb = estart + i * _B
            pltpu.sync_copy(src_hbm.at[pl.ds(eb, _B)], srcb)
            pltpu.sync_copy(dst_hbm.at[pl.ds(eb, _B)], dstb)
            pltpu.sync_copy(typ_hbm.at[pl.ds(eb, _B)], typb)
            for j in range(_B // _L):
                sl = pl.ds(j * _L, _L)
                gidx[sl] = srcb[sl] * _NCHUNK + k
                sidx[sl] = typb[sl] * _N + dstb[sl]
            pltpu.async_copy(xv_hbm.at[gidx], rows, sem).wait()
            pltpu.sync_copy(rows, acc.at[sidx], add=True)
            return carry

        lax.fori_loop(0, _EPT // _B, batch, None)
        plsc.subcore_barrier()
        pltpu.sync_copy(acc.at[pl.ds(base, _SPT)],
                        out_hbm.at[k, pl.ds(base, _SPT)])
        plsc.subcore_barrier()


_BN = 400


def _tc_dense(x, agg, cnt, w_rel, w_root, b):
    agg4 = agg.reshape(_NCHUNK, _R, _N, _L)
    cnt4 = cnt.reshape(_NC, _R, _N, _L)
    b2 = b.reshape(1, _D)

    def body(x_ref, agg_ref, cnt_ref, wrel_ref, wroot_ref, b_ref, o_ref):
        xb = x_ref[...]
        accv = jnp.dot(xb, wroot_ref[...],
                       preferred_element_type=jnp.float32) + b_ref[...]
        cntv = cnt_ref[0] + cnt_ref[1]
        inv = 1.0 / jnp.maximum(cntv, 1.0)
        for r in range(_R):
            m = jnp.concatenate(
                [agg_ref[kk, r] * inv[r] for kk in range(_NCHUNK)], axis=-1)
            accv = accv + jnp.dot(m, wrel_ref[r],
                                  preferred_element_type=jnp.float32)
        o_ref[...] = jnp.maximum(accv, 0.0)

    return pl.pallas_call(
        body,
        grid=(_N // _BN,),
        in_specs=[
            pl.BlockSpec((_BN, _D), lambda i: (i, 0)),
            pl.BlockSpec((_NCHUNK, _R, _BN, _L), lambda i: (0, 0, i, 0)),
            pl.BlockSpec((_NC, _R, _BN, _L), lambda i: (0, 0, i, 0)),
            pl.BlockSpec((_R, _D, _D), lambda i: (0, 0, 0)),
            pl.BlockSpec((_D, _D), lambda i: (0, 0)),
            pl.BlockSpec((1, _D), lambda i: (0, 0)),
        ],
        out_specs=pl.BlockSpec((_BN, _D), lambda i: (i, 0)),
        out_shape=jax.ShapeDtypeStruct((_N, _D), jnp.float32),
    )(x, agg4, cnt4, w_rel, w_root, b2)


def kernel(x, edge_index, edge_type, edge_attr,
           W_rel1, W_root1, b1, W_rel2, W_root2, b2):
    del edge_attr
    src = edge_index[0]
    dst = edge_index[1]
    cnt = _sc_count(dst, edge_type)
    agg1 = _sc_agg(x.reshape(_N * _NCHUNK, _L), src, dst, edge_type)
    h = _tc_dense(x, agg1, cnt, W_rel1, W_root1, b1)
    agg2 = _sc_agg(h.reshape(_N * _NCHUNK, _L), src, dst, edge_type)
    return _tc_dense(h, agg2, cnt, W_rel2, W_root2, b2)


# speedup vs baseline: 10.3111x; 3.0319x over previous
import functools

import jax
import jax.numpy as jnp
from jax import lax
from jax.experimental import pallas as pl
from jax.experimental.pallas import tpu as pltpu
from jax.experimental.pallas import tpu_sc as plsc

_N = 10000
_E = 320000
_D = 128
_R = 8
_L = 16
_NCHUNK = _D // _L
_NC = 2
_NS = 16
_CPC = _NCHUNK // _NC
_RN = _R * _N
_B = 80
_EPT = _E // _NS
_EPW = _E // (_NC * _NS)
_SPT = _RN // _NS
_ZR = 125
_TS = 2000

_mesh = plsc.VectorSubcoreMesh(core_axis_name="c", subcore_axis_name="s")
_sc_params = pltpu.CompilerParams(use_tc_tiling_on_sc=False)


def _fill(buf, nrows, value):
    def body(i, carry):
        buf[i, :] = jnp.full((_L,), value, jnp.float32)
        return carry
    lax.fori_loop(0, nrows, body, None)


def _clear_stripe(acc, zbuf, base, sem):
    n = _SPT // _ZR
    for q in range(n):
        pltpu.async_copy(zbuf, acc.at[pl.ds(base + q * _ZR, _ZR)], sem)
    for q in range(n):
        pltpu.make_async_copy(
            zbuf, acc.at[pl.ds(base + q * _ZR, _ZR)], sem).wait()


@functools.partial(
    pl.kernel,
    out_type=jax.ShapeDtypeStruct((_NC, _RN, _L), jnp.float32),
    mesh=_mesh,
    scratch_types=[
        pltpu.VMEM((_EPW,), jnp.int32),
        pltpu.VMEM((_EPW,), jnp.int32),
        pltpu.VMEM((_B, _L), jnp.float32),
        pltpu.VMEM((_ZR, _L), jnp.float32),
        pltpu.VMEM_SHARED((_RN, _L), jnp.float32),
        pltpu.SemaphoreType.DMA,
    ],
    compiler_params=_sc_params,
)
def _sc_count(dst_hbm, typ_hbm, out_hbm, dsta, typa, ones, zbuf, acc, sem):
    c = lax.axis_index("c")
    s = lax.axis_index("s")
    _fill(ones, _B, 1.0)
    _fill(zbuf, _ZR, 0.0)
    base = s * _SPT
    _clear_stripe(acc, zbuf, base, sem)

    estart = (c * _NS + s) * _EPW
    pltpu.sync_copy(dst_hbm.at[pl.ds(estart, _EPW)], dsta)
    pltpu.sync_copy(typ_hbm.at[pl.ds(estart, _EPW)], typa)

    def mkidx(i, carry):
        sl = pl.ds(i * _L, _L)
        dsta[sl] = typa[sl] * _N + dsta[sl]
        return carry

    lax.fori_loop(0, _EPW // _L, mkidx, None)
    plsc.subcore_barrier()

    def batch(i, carry):
        pltpu.sync_copy(ones, acc.at[dsta.at[pl.ds(i * _B, _B)]], add=True)
        return carry

    lax.fori_loop(0, _EPW // _B, batch, None)
    plsc.subcore_barrier()
    pltpu.sync_copy(acc.at[pl.ds(base, _SPT)], out_hbm.at[c, pl.ds(base, _SPT)])


@functools.partial(
    pl.kernel,
    out_type=jax.ShapeDtypeStruct((_NCHUNK, _RN, _L), jnp.float32),
    mesh=_mesh,
    scratch_types=[
        pltpu.VMEM((_EPT,), jnp.int32),
        pltpu.VMEM((_EPT,), jnp.int32),
        pltpu.VMEM((_TS,), jnp.int32),
        pltpu.VMEM((_B,), jnp.int32),
        pltpu.VMEM((_B,), jnp.int32),
        pltpu.VMEM((_B, _L), jnp.float32),
        pltpu.VMEM((_B, _L), jnp.float32),
        pltpu.VMEM((_ZR, _L), jnp.float32),
        pltpu.VMEM_SHARED((_RN, _L), jnp.float32),
        pltpu.SemaphoreType.DMA,
        pltpu.SemaphoreType.DMA,
    ],
    compiler_params=_sc_params,
)
def _sc_agg(xv_hbm, src_hbm, dst_hbm, typ_hbm, out_hbm,
            srca, sia, tps, gb_a, gb_b, rows_a, rows_b,
            zbuf, acc, sem_a, sem_b):
    c = lax.axis_index("c")
    s = lax.axis_index("s")
    _fill(zbuf, _ZR, 0.0)
    base = s * _SPT
    estart = s * _EPT
    pltpu.sync_copy(src_hbm.at[pl.ds(estart, _EPT)], srca)
    pltpu.sync_copy(dst_hbm.at[pl.ds(estart, _EPT)], sia)

    def mks(i, carry):
        sl = pl.ds(i * _L, _L)
        srca[sl] = srca[sl] * _NCHUNK
        return carry

    lax.fori_loop(0, _EPT // _L, mks, None)

    def typchunk(q, carry):
        pltpu.sync_copy(typ_hbm.at[pl.ds(estart + q * _TS, _TS)], tps)

        def inner(i, c2):
            esl = pl.ds(q * _TS + i * _L, _L)
            sia[esl] = tps[pl.ds(i * _L, _L)] * _N + sia[esl]
            return c2

        lax.fori_loop(0, _TS // _L, inner, None)
        return carry

    lax.fori_loop(0, _EPT // _TS, typchunk, None)

    nb = _EPT // _B

    def prep(b, gb, k):
        for j in range(_B // _L):
            sl = pl.ds(j * _L, _L)
            gb[sl] = srca[pl.ds(b * _B + j * _L, _L)] + k

    def gather(gb, rows, sem):
        return pltpu.async_copy(xv_hbm.at[gb], rows, sem)

    def scat(b, rows):
        pltpu.sync_copy(rows, acc.at[sia.at[pl.ds(b * _B, _B)]], add=True)

    for t in range(_CPC):
        k = c * _CPC + t
        _clear_stripe(acc, zbuf, base, sem_a)
        plsc.subcore_barrier()

        prep(0, gb_a, k)
        gather(gb_a, rows_a, sem_a)

        def body(i, carry):
            b0 = 2 * i
            prep(b0 + 1, gb_b, k)
            gather(gb_b, rows_b, sem_b)
            pltpu.make_async_copy(xv_hbm.at[gb_a], rows_a, sem_a).wait()
            scat(b0, rows_a)

            @pl.when(i < nb // 2 - 1)
            def _():
                prep(b0 + 2, gb_a, k)
                gather(gb_a, rows_a, sem_a)

            pltpu.make_async_copy(xv_hbm.at[gb_b], rows_b, sem_b).wait()
            scat(b0 + 1, rows_b)
            return carry

        lax.fori_loop(0, nb // 2, body, None)
        plsc.subcore_barrier()
        pltpu.sync_copy(acc.at[pl.ds(base, _SPT)],
                        out_hbm.at[k, pl.ds(base, _SPT)])


_BN = 400


def _tc_dense(x, agg, cnt, w_rel, w_root, b):
    agg4 = agg.reshape(_NCHUNK, _R, _N, _L)
    cnt4 = cnt.reshape(_NC, _R, _N, _L)
    b2 = b.reshape(1, _D)

    def body(x_ref, agg_ref, cnt_ref, wrel_ref, wroot_ref, b_ref, o_ref):
        xb = x_ref[...]
        accv = jnp.dot(xb, wroot_ref[...],
                       preferred_element_type=jnp.float32) + b_ref[...]
        cntv = cnt_ref[0] + cnt_ref[1]
        inv = 1.0 / jnp.maximum(cntv, 1.0)
        for r in range(_R):
            m = jnp.concatenate(
                [agg_ref[kk, r] * inv[r] for kk in range(_NCHUNK)], axis=-1)
            accv = accv + jnp.dot(m, wrel_ref[r],
                                  preferred_element_type=jnp.float32)
        o_ref[...] = jnp.maximum(accv, 0.0)

    return pl.pallas_call(
        body,
        grid=(_N // _BN,),
        in_specs=[
            pl.BlockSpec((_BN, _D), lambda i: (i, 0)),
            pl.BlockSpec((_NCHUNK, _R, _BN, _L), lambda i: (0, 0, i, 0)),
            pl.BlockSpec((_NC, _R, _BN, _L), lambda i: (0, 0, i, 0)),
            pl.BlockSpec((_R, _D, _D), lambda i: (0, 0, 0)),
            pl.BlockSpec((_D, _D), lambda i: (0, 0)),
            pl.BlockSpec((1, _D), lambda i: (0, 0)),
        ],
        out_specs=pl.BlockSpec((_BN, _D), lambda i: (i, 0)),
        out_shape=jax.ShapeDtypeStruct((_N, _D), jnp.float32),
    )(x, agg4, cnt4, w_rel, w_root, b2)


def kernel(x, edge_index, edge_type, edge_attr,
           W_rel1, W_root1, b1, W_rel2, W_root2, b2):
    del edge_attr
    src = edge_index[0]
    dst = edge_index[1]
    cnt = _sc_count(dst, edge_type)
    agg1 = _sc_agg(x.reshape(_N * _NCHUNK, _L), src, dst, edge_type)
    h = _tc_dense(x, agg1, cnt, W_rel1, W_root1, b1)
    agg2 = _sc_agg(h.reshape(_N * _NCHUNK, _L), src, dst, edge_type)
    return _tc_dense(h, agg2, cnt, W_rel2, W_root2, b2)


# speedup vs baseline: 10.3179x; 1.0007x over previous
import functools

import jax
import jax.numpy as jnp
from jax import lax
from jax.experimental import pallas as pl
from jax.experimental.pallas import tpu as pltpu
from jax.experimental.pallas import tpu_sc as plsc

_N = 10000
_E = 320000
_D = 128
_R = 8
_L = 16
_NCHUNK = _D // _L
_NC = 2
_NS = 16
_CPC = _NCHUNK // _NC
_RN = _R * _N
_B = 80
_EPT = _E // _NS
_EPW = _E // (_NC * _NS)
_SPT = _RN // _NS
_ZR = 125
_TS = 2000

_mesh = plsc.VectorSubcoreMesh(core_axis_name="c", subcore_axis_name="s")
_sc_params = pltpu.CompilerParams(use_tc_tiling_on_sc=False,
                                  skip_device_barrier=True)


def _fill(buf, nrows, value):
    def body(i, carry):
        buf[i, :] = jnp.full((_L,), value, jnp.float32)
        return carry
    lax.fori_loop(0, nrows, body, None)


def _clear_stripe(acc, zbuf, base, sem):
    n = _SPT // _ZR
    for q in range(n):
        pltpu.async_copy(zbuf, acc.at[pl.ds(base + q * _ZR, _ZR)], sem)
    for q in range(n):
        pltpu.make_async_copy(
            zbuf, acc.at[pl.ds(base + q * _ZR, _ZR)], sem).wait()


@functools.partial(
    pl.kernel,
    out_type=jax.ShapeDtypeStruct((_NC, _RN, _L), jnp.float32),
    mesh=_mesh,
    scratch_types=[
        pltpu.VMEM((_EPW,), jnp.int32),
        pltpu.VMEM((_EPW,), jnp.int32),
        pltpu.VMEM((_B, _L), jnp.float32),
        pltpu.VMEM((_ZR, _L), jnp.float32),
        pltpu.VMEM_SHARED((_RN, _L), jnp.float32),
        pltpu.SemaphoreType.DMA,
    ],
    compiler_params=_sc_params,
)
def _sc_count(dst_hbm, typ_hbm, out_hbm, dsta, typa, ones, zbuf, acc, sem):
    c = lax.axis_index("c")
    s = lax.axis_index("s")
    _fill(ones, _B, 1.0)
    _fill(zbuf, _ZR, 0.0)
    base = s * _SPT
    _clear_stripe(acc, zbuf, base, sem)

    estart = (c * _NS + s) * _EPW
    pltpu.sync_copy(dst_hbm.at[pl.ds(estart, _EPW)], dsta)
    pltpu.sync_copy(typ_hbm.at[pl.ds(estart, _EPW)], typa)

    def mkidx(i, carry):
        sl = pl.ds(i * _L, _L)
        dsta[sl] = typa[sl] * _N + dsta[sl]
        return carry

    lax.fori_loop(0, _EPW // _L, mkidx, None)
    plsc.subcore_barrier()

    def batch(i, carry):
        pltpu.sync_copy(ones, acc.at[dsta.at[pl.ds(i * _B, _B)]], add=True)
        return carry

    lax.fori_loop(0, _EPW // _B, batch, None)
    plsc.subcore_barrier()
    pltpu.sync_copy(acc.at[pl.ds(base, _SPT)], out_hbm.at[c, pl.ds(base, _SPT)])


@functools.partial(
    pl.kernel,
    out_type=jax.ShapeDtypeStruct((_NCHUNK, _RN, _L), jnp.float32),
    mesh=_mesh,
    scratch_types=[
        pltpu.VMEM((_EPT,), jnp.int32),
        pltpu.VMEM((_EPT,), jnp.int32),
        pltpu.VMEM((_TS,), jnp.int32),
        pltpu.VMEM((_B,), jnp.int32),
        pltpu.VMEM((_B,), jnp.int32),
        pltpu.VMEM((_B, _L), jnp.float32),
        pltpu.VMEM((_B, _L), jnp.float32),
        pltpu.VMEM((_ZR, _L), jnp.float32),
        pltpu.VMEM_SHARED((_RN, _L), jnp.float32),
        pltpu.SemaphoreType.DMA,
        pltpu.SemaphoreType.DMA,
    ],
    compiler_params=_sc_params,
)
def _sc_agg(xv_hbm, src_hbm, dst_hbm, typ_hbm, out_hbm,
            srca, sia, tps, gb_a, gb_b, rows_a, rows_b,
            zbuf, acc, sem_a, sem_b):
    c = lax.axis_index("c")
    s = lax.axis_index("s")
    _fill(zbuf, _ZR, 0.0)
    base = s * _SPT
    estart = s * _EPT
    pltpu.sync_copy(src_hbm.at[pl.ds(estart, _EPT)], srca)
    pltpu.sync_copy(dst_hbm.at[pl.ds(estart, _EPT)], sia)

    def mks(i, carry):
        sl = pl.ds(i * _L, _L)
        srca[sl] = srca[sl] * _NCHUNK
        return carry

    lax.fori_loop(0, _EPT // _L, mks, None)

    def typchunk(q, carry):
        pltpu.sync_copy(typ_hbm.at[pl.ds(estart + q * _TS, _TS)], tps)

        def inner(i, c2):
            esl = pl.ds(q * _TS + i * _L, _L)
            sia[esl] = tps[pl.ds(i * _L, _L)] * _N + sia[esl]
            return c2

        lax.fori_loop(0, _TS // _L, inner, None)
        return carry

    lax.fori_loop(0, _EPT // _TS, typchunk, None)

    nb = _EPT // _B

    def prep(b, gb, k):
        for j in range(_B // _L):
            sl = pl.ds(j * _L, _L)
            gb[sl] = srca[pl.ds(b * _B + j * _L, _L)] + k

    def gather(gb, rows, sem):
        return pltpu.async_copy(xv_hbm.at[gb], rows, sem)

    def scat(b, rows):
        pltpu.sync_copy(rows, acc.at[sia.at[pl.ds(b * _B, _B)]], add=True)

    for t in range(_CPC):
        k = c * _CPC + t
        _clear_stripe(acc, zbuf, base, sem_a)
        plsc.subcore_barrier()

        prep(0, gb_a, k)
        gather(gb_a, rows_a, sem_a)

        def body(i, carry):
            b0 = 2 * i
            prep(b0 + 1, gb_b, k)
            gather(gb_b, rows_b, sem_b)
            pltpu.make_async_copy(xv_hbm.at[gb_a], rows_a, sem_a).wait()
            scat(b0, rows_a)

            @pl.when(i < nb // 2 - 1)
            def _():
                prep(b0 + 2, gb_a, k)
                gather(gb_a, rows_a, sem_a)

            pltpu.make_async_copy(xv_hbm.at[gb_b], rows_b, sem_b).wait()
            scat(b0 + 1, rows_b)
            return carry

        lax.fori_loop(0, nb // 2, body, None)
        plsc.subcore_barrier()
        pltpu.sync_copy(acc.at[pl.ds(base, _SPT)],
                        out_hbm.at[k, pl.ds(base, _SPT)])


_BN = 400


def _tc_dense(x, agg, cnt, w_rel, w_root, b):
    agg4 = agg.reshape(_NCHUNK, _R, _N, _L)
    cnt4 = cnt.reshape(_NC, _R, _N, _L)
    b2 = b.reshape(1, _D)

    def body(x_ref, agg_ref, cnt_ref, wrel_ref, wroot_ref, b_ref, o_ref):
        xb = x_ref[...]
        accv = jnp.dot(xb, wroot_ref[...],
                       preferred_element_type=jnp.float32) + b_ref[...]
        cntv = cnt_ref[0] + cnt_ref[1]
        inv = 1.0 / jnp.maximum(cntv, 1.0)
        for r in range(_R):
            m = jnp.concatenate(
                [agg_ref[kk, r] * inv[r] for kk in range(_NCHUNK)], axis=-1)
            accv = accv + jnp.dot(m, wrel_ref[r],
                                  preferred_element_type=jnp.float32)
        o_ref[...] = jnp.maximum(accv, 0.0)

    return pl.pallas_call(
        body,
        grid=(_N // _BN,),
        in_specs=[
            pl.BlockSpec((_BN, _D), lambda i: (i, 0)),
            pl.BlockSpec((_NCHUNK, _R, _BN, _L), lambda i: (0, 0, i, 0)),
            pl.BlockSpec((_NC, _R, _BN, _L), lambda i: (0, 0, i, 0)),
            pl.BlockSpec((_R, _D, _D), lambda i: (0, 0, 0)),
            pl.BlockSpec((_D, _D), lambda i: (0, 0)),
            pl.BlockSpec((1, _D), lambda i: (0, 0)),
        ],
        out_specs=pl.BlockSpec((_BN, _D), lambda i: (i, 0)),
        out_shape=jax.ShapeDtypeStruct((_N, _D), jnp.float32),
    )(x, agg4, cnt4, w_rel, w_root, b2)


def kernel(x, edge_index, edge_type, edge_attr,
           W_rel1, W_root1, b1, W_rel2, W_root2, b2):
    del edge_attr
    src = edge_index[0]
    dst = edge_index[1]
    cnt = _sc_count(dst, edge_type)
    agg1 = _sc_agg(x.reshape(_N * _NCHUNK, _L), src, dst, edge_type)
    h = _tc_dense(x, agg1, cnt, W_rel1, W_root1, b1)
    agg2 = _sc_agg(h.reshape(_N * _NCHUNK, _L), src, dst, edge_type)
    return _tc_dense(h, agg2, cnt, W_rel2, W_root2, b2)


# speedup vs baseline: 12.5602x; 1.2173x over previous
import functools

import jax
import jax.numpy as jnp
from jax import lax
from jax.experimental import pallas as pl
from jax.experimental.pallas import tpu as pltpu
from jax.experimental.pallas import tpu_sc as plsc

_N = 10000
_E = 320000
_D = 128
_R = 8
_L = 16
_NCHUNK = _D // _L
_NC = 2
_NS = 16
_CPC = _NCHUNK // _NC
_RN = _R * _N
_B = 128
_EPT = _E // _NS
_EPW = _E // (_NC * _NS)
_SPT = _RN // _NS
_ZR = 125
_NB = _EPT // _B
_TAIL = _EPT - _NB * _B
_CNB = _EPW // _B
_CTAIL = _EPW - _CNB * _B

_mesh = plsc.VectorSubcoreMesh(core_axis_name="c", subcore_axis_name="s")
_sc_params = pltpu.CompilerParams(use_tc_tiling_on_sc=False,
                                  skip_device_barrier=True)


def _fill(buf, nrows, value):
    def body(i, carry):
        buf[i, :] = jnp.full((_L,), value, jnp.float32)
        return carry
    lax.fori_loop(0, nrows, body, None)


def _clear_stripe(acc, zbuf, base, sem):
    n = _SPT // _ZR
    for q in range(n):
        pltpu.async_copy(zbuf, acc.at[pl.ds(base + q * _ZR, _ZR)], sem)
    for q in range(n):
        pltpu.make_async_copy(
            zbuf, acc.at[pl.ds(base + q * _ZR, _ZR)], sem).wait()


def _sc_agg_body(with_count, xv_hbm, src_hbm, dst_hbm, typ_hbm, *rest):
    if with_count:
        (agg_out, cnt_out, srca, sia, gb0, gb1, gb2, rw0, rw1, rw2,
         zbuf, acc, gs0, gs1, gs2, ss0, ss1, ss2) = rest
    else:
        (agg_out, srca, sia, gb0, gb1, gb2, rw0, rw1, rw2,
         zbuf, acc, gs0, gs1, gs2, ss0, ss1, ss2) = rest
    gbs = (gb0, gb1, gb2)
    rws = (rw0, rw1, rw2)
    gss = (gs0, gs1, gs2)
    sss = (ss0, ss1, ss2)

    c = lax.axis_index("c")
    s = lax.axis_index("s")
    base = s * _SPT
    estart = s * _EPT
    _fill(zbuf, _ZR, 0.0)

    pltpu.sync_copy(dst_hbm.at[pl.ds(estart, _EPT)], sia)
    pltpu.sync_copy(typ_hbm.at[pl.ds(estart, _EPT)], srca)

    def mksi(i, carry):
        sl = pl.ds(i * _L, _L)
        sia[sl] = srca[sl] * _N + sia[sl]
        return carry

    lax.fori_loop(0, _EPT // _L, mksi, None)
    pltpu.sync_copy(src_hbm.at[pl.ds(estart, _EPT)], srca)

    def mkg(i, carry):
        sl = pl.ds(i * _L, _L)
        srca[sl] = srca[sl] * _NCHUNK
        return carry

    lax.fori_loop(0, _EPT // _L, mkg, None)

    if with_count:
        _fill(rw0, _B, 1.0)
        _clear_stripe(acc, zbuf, base, gs0)
        plsc.subcore_barrier()
        cstart = c * _EPW

        def cbatch(i, carry):
            pltpu.sync_copy(
                rw0, acc.at[sia.at[pl.ds(cstart + i * _B, _B)]], add=True)
            return carry

        lax.fori_loop(0, _CNB, cbatch, None)
        pltpu.sync_copy(
            rw0.at[pl.ds(0, _CTAIL)],
            acc.at[sia.at[pl.ds(cstart + _CNB * _B, _CTAIL)]], add=True)
        plsc.subcore_barrier()
        pltpu.sync_copy(acc.at[pl.ds(base, _SPT)],
                        cnt_out.at[c, pl.ds(base, _SPT)])

    def prep(b, gb, k):
        for j in range(_B // _L):
            gb[pl.ds(j * _L, _L)] = srca[pl.ds(b * _B + j * _L, _L)] + k

    def gath(gb, rw, sem):
        pltpu.async_copy(xv_hbm.at[gb], rw, sem)

    def gwait(gb, rw, sem):
        pltpu.make_async_copy(xv_hbm.at[gb], rw, sem).wait()

    def sstart(b, rw, sem):
        pltpu.async_copy(rw, acc.at[sia.at[pl.ds(b * _B, _B)]], sem, add=True)

    def swait(b, rw, sem):
        pltpu.make_async_copy(
            rw, acc.at[sia.at[pl.ds(b * _B, _B)]], sem).wait()

    for t in range(_CPC):
        k = c * _CPC + t
        _clear_stripe(acc, zbuf, base, gs0)
        plsc.subcore_barrier()

        prep(0, gb0, k)
        gath(gb0, rw0, gs0)
        prep(1, gb1, k)
        gath(gb1, rw1, gs1)

        def slot(b, bi, first):
            x2 = (bi + 2) % 3
            gwait(gbs[bi], rws[bi], gss[bi])
            sstart(b, rws[bi], sss[bi])
            if first:
                prep(b + 2, gbs[x2], k)
                gath(gbs[x2], rws[x2], gss[x2])
            else:
                @pl.when(b + 2 < _NB)
                def _():
                    prep(b + 2, gbs[x2], k)
                    swait(b - 1, rws[x2], sss[x2])
                    gath(gbs[x2], rws[x2], gss[x2])

        slot(0, 0, True)
        slot(1, 1, False)
        slot(2, 2, False)

        def body(i, carry):
            b0 = 3 * i
            slot(b0, 0, False)
            slot(b0 + 1, 1, False)
            slot(b0 + 2, 2, False)
            return carry

        lax.fori_loop(1, _NB // 3, body, None)
        swait(_NB - 3, rws[0], sss[0])
        swait(_NB - 2, rws[1], sss[1])
        swait(_NB - 1, rws[2], sss[2])
        for j in range(_TAIL // _L):
            gb0[pl.ds(j * _L, _L)] = (
                srca[pl.ds(_NB * _B + j * _L, _L)] + k)
        pltpu.async_copy(
            xv_hbm.at[gb0.at[pl.ds(0, _TAIL)]],
            rw0.at[pl.ds(0, _TAIL)], gs0)
        pltpu.make_async_copy(
            xv_hbm.at[gb0.at[pl.ds(0, _TAIL)]],
            rw0.at[pl.ds(0, _TAIL)], gs0).wait()
        pltpu.sync_copy(
            rw0.at[pl.ds(0, _TAIL)],
            acc.at[sia.at[pl.ds(_NB * _B, _TAIL)]], add=True)

        plsc.subcore_barrier()
        pltpu.sync_copy(acc.at[pl.ds(base, _SPT)],
                        agg_out.at[k, pl.ds(base, _SPT)])
        if t < _CPC - 1:
            plsc.subcore_barrier()


def _make_sc_agg(with_count):
    out_types = [jax.ShapeDtypeStruct((_NCHUNK, _RN, _L), jnp.float32)]
    if with_count:
        out_types.append(jax.ShapeDtypeStruct((_NC, _RN, _L), jnp.float32))
    scratch = [
        pltpu.VMEM((_EPT,), jnp.int32),
        pltpu.VMEM((_EPT,), jnp.int32),
        pltpu.VMEM((_B,), jnp.int32),
        pltpu.VMEM((_B,), jnp.int32),
        pltpu.VMEM((_B,), jnp.int32),
        pltpu.VMEM((_B, _L), jnp.float32),
        pltpu.VMEM((_B, _L), jnp.float32),
        pltpu.VMEM((_B, _L), jnp.float32),
        pltpu.VMEM((_ZR, _L), jnp.float32),
        pltpu.VMEM_SHARED((_RN, _L), jnp.float32),
        pltpu.SemaphoreType.DMA,
        pltpu.SemaphoreType.DMA,
        pltpu.SemaphoreType.DMA,
        pltpu.SemaphoreType.DMA,
        pltpu.SemaphoreType.DMA,
        pltpu.SemaphoreType.DMA,
    ]
    return pl.kernel(
        functools.partial(_sc_agg_body, with_count),
        out_type=tuple(out_types) if with_count else out_types[0],
        mesh=_mesh,
        scratch_types=scratch,
        compiler_params=_sc_params,
    )


_sc_agg_cnt = _make_sc_agg(True)
_sc_agg = _make_sc_agg(False)


_BN = 400


def _tc_dense(x, agg, cnt, w_rel, w_root, b):
    agg4 = agg.reshape(_NCHUNK, _R, _N, _L)
    cnt4 = cnt.reshape(_NC, _R, _N, _L)
    b2 = b.reshape(1, _D)

    def body(x_ref, agg_ref, cnt_ref, wrel_ref, wroot_ref, b_ref, o_ref):
        xb = x_ref[...]
        accv = jnp.dot(xb, wroot_ref[...],
                       preferred_element_type=jnp.float32) + b_ref[...]
        cntv = cnt_ref[0] + cnt_ref[1]
        inv = 1.0 / jnp.maximum(cntv, 1.0)
        for r in range(_R):
            m = jnp.concatenate(
                [agg_ref[kk, r] * inv[r] for kk in range(_NCHUNK)], axis=-1)
            accv = accv + jnp.dot(m, wrel_ref[r],
                                  preferred_element_type=jnp.float32)
        o_ref[...] = jnp.maximum(accv, 0.0)

    return pl.pallas_call(
        body,
        grid=(_N // _BN,),
        in_specs=[
            pl.BlockSpec((_BN, _D), lambda i: (i, 0)),
            pl.BlockSpec((_NCHUNK, _R, _BN, _L), lambda i: (0, 0, i, 0)),
            pl.BlockSpec((_NC, _R, _BN, _L), lambda i: (0, 0, i, 0)),
            pl.BlockSpec((_R, _D, _D), lambda i: (0, 0, 0)),
            pl.BlockSpec((_D, _D), lambda i: (0, 0)),
            pl.BlockSpec((1, _D), lambda i: (0, 0)),
        ],
        out_specs=pl.BlockSpec((_BN, _D), lambda i: (i, 0)),
        out_shape=jax.ShapeDtypeStruct((_N, _D), jnp.float32),
    )(x, agg4, cnt4, w_rel, w_root, b2)


def kernel(x, edge_index, edge_type, edge_attr,
           W_rel1, W_root1, b1, W_rel2, W_root2, b2):
    del edge_attr
    src = edge_index[0]
    dst = edge_index[1]
    agg1, cnt = _sc_agg_cnt(x.reshape(_N * _NCHUNK, _L), src, dst, edge_type)
    h = _tc_dense(x, agg1, cnt, W_rel1, W_root1, b1)
    agg2 = _sc_agg(h.reshape(_N * _NCHUNK, _L), src, dst, edge_type)
    return _tc_dense(h, agg2, cnt, W_rel2, W_root2, b2)


# speedup vs baseline: 16.9052x; 1.3459x over previous
import functools

import jax
import jax.numpy as jnp
from jax import lax
from jax.experimental import pallas as pl
from jax.experimental.pallas import tpu as pltpu
from jax.experimental.pallas import tpu_sc as plsc

_N = 10000
_E = 320000
_D = 128
_R = 8
_L = 16
_NCHUNK = _D // _L
_NC = 2
_NS = 16
_CPC = _NCHUNK // _NC
_RN = _R * _N
_B = 128
_EPT = _E // _NS
_EPW = _E // (_NC * _NS)
_SPT = _RN // _NS
_ZR = 125
_NB = _EPT // _B
_TAIL = _EPT - _NB * _B
_CNB = _EPW // _B
_CTAIL = _EPW - _CNB * _B

_mesh = plsc.VectorSubcoreMesh(core_axis_name="c", subcore_axis_name="s")
_sc_params = pltpu.CompilerParams(use_tc_tiling_on_sc=False,
                                  skip_device_barrier=True)


def _fill(buf, nrows, value):
    def body(i, carry):
        buf[i, :] = jnp.full((_L,), value, jnp.float32)
        return carry
    lax.fori_loop(0, nrows, body, None)


def _clear_stripe(acc, zbuf, base, sem):
    n = _SPT // _ZR
    for q in range(n):
        pltpu.async_copy(zbuf, acc.at[pl.ds(base + q * _ZR, _ZR)], sem)
    for q in range(n):
        pltpu.make_async_copy(
            zbuf, acc.at[pl.ds(base + q * _ZR, _ZR)], sem).wait()


def _sc_agg_body(with_count, xv_hbm, src_hbm, dst_hbm, typ_hbm, *rest):
    if with_count:
        (agg_out, cnt_out, srca, sia, gb0, gb1, gb2, rw0, rw1, rw2,
         zbuf, acc, gs0, gs1, gs2, ss0, ss1, ss2) = rest
    else:
        (agg_out, srca, sia, gb0, gb1, gb2, rw0, rw1, rw2,
         zbuf, acc, gs0, gs1, gs2, ss0, ss1, ss2) = rest
    gbs = (gb0, gb1, gb2)
    rws = (rw0, rw1, rw2)
    gss = (gs0, gs1, gs2)
    sss = (ss0, ss1, ss2)

    c = lax.axis_index("c")
    s = lax.axis_index("s")
    base = s * _SPT
    estart = s * _EPT
    _fill(zbuf, _ZR, 0.0)

    pltpu.sync_copy(dst_hbm.at[pl.ds(estart, _EPT)], sia)
    pltpu.sync_copy(typ_hbm.at[pl.ds(estart, _EPT)], srca)

    def mksi(i, carry):
        sl = pl.ds(i * _L, _L)
        sia[sl] = srca[sl] * _N + sia[sl]
        return carry

    lax.fori_loop(0, _EPT // _L, mksi, None)
    pltpu.sync_copy(src_hbm.at[pl.ds(estart, _EPT)], srca)

    def mkg(i, carry):
        sl = pl.ds(i * _L, _L)
        srca[sl] = srca[sl] * _NCHUNK
        return carry

    lax.fori_loop(0, _EPT // _L, mkg, None)

    if with_count:
        _fill(rw0, _B, 1.0)
        _clear_stripe(acc, zbuf, base, gs0)
        plsc.subcore_barrier()
        cstart = c * _EPW

        def cbatch(i, carry):
            pltpu.sync_copy(
                rw0, acc.at[sia.at[pl.ds(cstart + i * _B, _B)]], add=True)
            return carry

        lax.fori_loop(0, _CNB, cbatch, None)
        pltpu.sync_copy(
            rw0.at[pl.ds(0, _CTAIL)],
            acc.at[sia.at[pl.ds(cstart + _CNB * _B, _CTAIL)]], add=True)
        plsc.subcore_barrier()
        pltpu.sync_copy(acc.at[pl.ds(base, _SPT)],
                        cnt_out.at[c, pl.ds(base, _SPT)])

    def prep(b, gb, k):
        for j in range(_B // _L):
            gb[pl.ds(j * _L, _L)] = srca[pl.ds(b * _B + j * _L, _L)] + k

    def gath(gb, rw, sem):
        pltpu.async_copy(xv_hbm.at[gb], rw, sem)

    def gwait(gb, rw, sem):
        pltpu.make_async_copy(xv_hbm.at[gb], rw, sem).wait()

    def sstart(b, rw, sem):
        pltpu.async_copy(rw, acc.at[sia.at[pl.ds(b * _B, _B)]], sem, add=True)

    def swait(b, rw, sem):
        pltpu.make_async_copy(
            rw, acc.at[sia.at[pl.ds(b * _B, _B)]], sem).wait()

    for t in range(_CPC):
        k = c * _CPC + t
        _clear_stripe(acc, zbuf, base, gs0)
        plsc.subcore_barrier()

        prep(0, gb0, k)
        gath(gb0, rw0, gs0)
        prep(1, gb1, k)
        gath(gb1, rw1, gs1)

        def slot(b, bi, first):
            x2 = (bi + 2) % 3
            gwait(gbs[bi], rws[bi], gss[bi])
            sstart(b, rws[bi], sss[bi])
            if first:
                prep(b + 2, gbs[x2], k)
                gath(gbs[x2], rws[x2], gss[x2])
            else:
                @pl.when(b + 2 < _NB)
                def _():
                    prep(b + 2, gbs[x2], k)
                    swait(b - 1, rws[x2], sss[x2])
                    gath(gbs[x2], rws[x2], gss[x2])

        slot(0, 0, True)
        slot(1, 1, False)
        slot(2, 2, False)

        def body(i, carry):
            b0 = 3 * i
            slot(b0, 0, False)
            slot(b0 + 1, 1, False)
            slot(b0 + 2, 2, False)
            return carry

        lax.fori_loop(1, _NB // 3, body, None)
        swait(_NB - 3, rws[0], sss[0])
        swait(_NB - 2, rws[1], sss[1])
        swait(_NB - 1, rws[2], sss[2])
        for j in range(_TAIL // _L):
            gb0[pl.ds(j * _L, _L)] = (
                srca[pl.ds(_NB * _B + j * _L, _L)] + k)
        pltpu.async_copy(
            xv_hbm.at[gb0.at[pl.ds(0, _TAIL)]],
            rw0.at[pl.ds(0, _TAIL)], gs0)
        pltpu.make_async_copy(
            xv_hbm.at[gb0.at[pl.ds(0, _TAIL)]],
            rw0.at[pl.ds(0, _TAIL)], gs0).wait()
        pltpu.sync_copy(
            rw0.at[pl.ds(0, _TAIL)],
            acc.at[sia.at[pl.ds(_NB * _B, _TAIL)]], add=True)

        plsc.subcore_barrier()
        pltpu.sync_copy(acc.at[pl.ds(base, _SPT)],
                        agg_out.at[pl.ds(base, _SPT), k])
        if t < _CPC - 1:
            plsc.subcore_barrier()


def _make_sc_agg(with_count):
    out_types = [jax.ShapeDtypeStruct((_RN, _NCHUNK, _L), jnp.float32)]
    if with_count:
        out_types.append(jax.ShapeDtypeStruct((_NC, _RN, _L), jnp.float32))
    scratch = [
        pltpu.VMEM((_EPT,), jnp.int32),
        pltpu.VMEM((_EPT,), jnp.int32),
        pltpu.VMEM((_B,), jnp.int32),
        pltpu.VMEM((_B,), jnp.int32),
        pltpu.VMEM((_B,), jnp.int32),
        pltpu.VMEM((_B, _L), jnp.float32),
        pltpu.VMEM((_B, _L), jnp.float32),
        pltpu.VMEM((_B, _L), jnp.float32),
        pltpu.VMEM((_ZR, _L), jnp.float32),
        pltpu.VMEM_SHARED((_RN, _L), jnp.float32),
        pltpu.SemaphoreType.DMA,
        pltpu.SemaphoreType.DMA,
        pltpu.SemaphoreType.DMA,
        pltpu.SemaphoreType.DMA,
        pltpu.SemaphoreType.DMA,
        pltpu.SemaphoreType.DMA,
    ]
    return pl.kernel(
        functools.partial(_sc_agg_body, with_count),
        out_type=tuple(out_types) if with_count else out_types[0],
        mesh=_mesh,
        scratch_types=scratch,
        compiler_params=_sc_params,
    )


_sc_agg_cnt = _make_sc_agg(True)
_sc_agg = _make_sc_agg(False)


_BN = 400


def _tc_dense(x, agg, cnt, w_rel, w_root, b):
    agg3 = agg.reshape(_R, _N, _D)
    cnt4 = cnt.reshape(_NC, _R, _N, _L)
    b2 = b.reshape(1, _D)

    def body(x_ref, agg_ref, cnt_ref, wrel_ref, wroot_ref, b_ref, o_ref):
        xb = x_ref[...]
        accv = jnp.dot(xb, wroot_ref[...],
                       preferred_element_type=jnp.float32) + b_ref[...]
        cntv = cnt_ref[0] + cnt_ref[1]
        inv = 1.0 / jnp.maximum(cntv[:, :, :1], 1.0)
        for r in range(_R):
            m = agg_ref[r] * inv[r]
            accv = accv + jnp.dot(m, wrel_ref[r],
                                  preferred_element_type=jnp.float32)
        o_ref[...] = jnp.maximum(accv, 0.0)

    return pl.pallas_call(
        body,
        grid=(_N // _BN,),
        in_specs=[
            pl.BlockSpec((_BN, _D), lambda i: (i, 0)),
            pl.BlockSpec((_R, _BN, _D), lambda i: (0, i, 0)),
            pl.BlockSpec((_NC, _R, _BN, _L), lambda i: (0, 0, i, 0)),
            pl.BlockSpec((_R, _D, _D), lambda i: (0, 0, 0)),
            pl.BlockSpec((_D, _D), lambda i: (0, 0)),
            pl.BlockSpec((1, _D), lambda i: (0, 0)),
        ],
        out_specs=pl.BlockSpec((_BN, _D), lambda i: (i, 0)),
        out_shape=jax.ShapeDtypeStruct((_N, _D), jnp.float32),
    )(x, agg3, cnt4, w_rel, w_root, b2)


def kernel(x, edge_index, edge_type, edge_attr,
           W_rel1, W_root1, b1, W_rel2, W_root2, b2):
    del edge_attr
    src = edge_index[0]
    dst = edge_index[1]
    agg1, cnt = _sc_agg_cnt(x.reshape(_N * _NCHUNK, _L), src, dst, edge_type)
    h = _tc_dense(x, agg1, cnt, W_rel1, W_root1, b1)
    agg2 = _sc_agg(h.reshape(_N * _NCHUNK, _L), src, dst, edge_type)
    return _tc_dense(h, agg2, cnt, W_rel2, W_root2, b2)


# speedup vs baseline: 16.9702x; 1.0038x over previous
import functools

import jax
import jax.numpy as jnp
from jax import lax
from jax.experimental import pallas as pl
from jax.experimental.pallas import tpu as pltpu
from jax.experimental.pallas import tpu_sc as plsc

_N = 10000
_E = 320000
_D = 128
_R = 8
_L = 16
_NCHUNK = _D // _L
_NC = 2
_NS = 16
_CPC = _NCHUNK // _NC
_RN = _R * _N
_B = 128
_EPT = _E // _NS
_EPW = _E // (_NC * _NS)
_SPT = _RN // _NS
_ZR = 125
_NB = _EPT // _B
_TAIL = _EPT - _NB * _B
_CNB = _EPW // _B
_CTAIL = _EPW - _CNB * _B

_mesh = plsc.VectorSubcoreMesh(core_axis_name="c", subcore_axis_name="s")
_sc_params = pltpu.CompilerParams(use_tc_tiling_on_sc=False,
                                  skip_device_barrier=True)


def _fill(buf, nrows, value):
    def body(i, carry):
        buf[i, :] = jnp.full((_L,), value, jnp.float32)
        return carry
    lax.fori_loop(0, nrows, body, None)


def _clear_stripe(acc, zbuf, base, sem):
    n = _SPT // _ZR
    for q in range(n):
        pltpu.async_copy(zbuf, acc.at[pl.ds(base + q * _ZR, _ZR)], sem)
    for q in range(n):
        pltpu.make_async_copy(
            zbuf, acc.at[pl.ds(base + q * _ZR, _ZR)], sem).wait()


def _sc_agg_body(with_count, xv_hbm, src_hbm, dst_hbm, typ_hbm, *rest):
    if with_count:
        (agg_out, cnt_out, srca, sia, gb0, gb1, gb2, rw0, rw1, rw2,
         zbuf, acc, gs0, gs1, gs2, ss0, ss1, ss2) = rest
    else:
        (agg_out, srca, sia, gb0, gb1, gb2, rw0, rw1, rw2,
         zbuf, acc, gs0, gs1, gs2, ss0, ss1, ss2) = rest
    gbs = (gb0, gb1, gb2)
    rws = (rw0, rw1, rw2)
    gss = (gs0, gs1, gs2)
    sss = (ss0, ss1, ss2)

    c = lax.axis_index("c")
    s = lax.axis_index("s")
    base = s * _SPT
    estart = s * _EPT
    _fill(zbuf, _ZR, 0.0)

    pltpu.sync_copy(dst_hbm.at[pl.ds(estart, _EPT)], sia)
    pltpu.sync_copy(typ_hbm.at[pl.ds(estart, _EPT)], srca)

    def mksi(i, carry):
        sl = pl.ds(i * _L, _L)
        sia[sl] = srca[sl] * _N + sia[sl]
        return carry

    lax.fori_loop(0, _EPT // _L, mksi, None)
    pltpu.sync_copy(src_hbm.at[pl.ds(estart, _EPT)], srca)

    def mkg(i, carry):
        sl = pl.ds(i * _L, _L)
        srca[sl] = srca[sl] * _NCHUNK
        return carry

    lax.fori_loop(0, _EPT // _L, mkg, None)

    if with_count:
        _fill(rw0, _B, 1.0)
        _clear_stripe(acc, zbuf, base, gs0)
        plsc.subcore_barrier()
        cstart = c * _EPW

        def cbatch(i, carry):
            pltpu.sync_copy(
                rw0, acc.at[sia.at[pl.ds(cstart + i * _B, _B)]], add=True)
            return carry

        lax.fori_loop(0, _CNB, cbatch, None)
        pltpu.sync_copy(
            rw0.at[pl.ds(0, _CTAIL)],
            acc.at[sia.at[pl.ds(cstart + _CNB * _B, _CTAIL)]], add=True)
        plsc.subcore_barrier()
        pltpu.sync_copy(acc.at[pl.ds(base, _SPT)],
                        cnt_out.at[c, pl.ds(base, _SPT)])

    def prep(b, gb, k):
        for j in range(_B // _L):
            gb[pl.ds(j * _L, _L)] = srca[pl.ds(b * _B + j * _L, _L)] + k

    def gath(gb, rw, sem):
        pltpu.async_copy(xv_hbm.at[gb], rw, sem)

    def gwait(gb, rw, sem):
        pltpu.make_async_copy(xv_hbm.at[gb], rw, sem).wait()

    def sstart(b, rw, sem):
        pltpu.async_copy(rw, acc.at[sia.at[pl.ds(b * _B, _B)]], sem, add=True)

    def swait(b, rw, sem):
        pltpu.make_async_copy(
            rw, acc.at[sia.at[pl.ds(b * _B, _B)]], sem).wait()

    def chunk_pass(t, carry):
        k = c * _CPC + t
        _clear_stripe(acc, zbuf, base, gs0)
        plsc.subcore_barrier()

        prep(0, gb0, k)
        gath(gb0, rw0, gs0)
        prep(1, gb1, k)
        gath(gb1, rw1, gs1)

        def slot(b, bi, first):
            x2 = (bi + 2) % 3
            gwait(gbs[bi], rws[bi], gss[bi])
            sstart(b, rws[bi], sss[bi])
            if first:
                prep(b + 2, gbs[x2], k)
                gath(gbs[x2], rws[x2], gss[x2])
            else:
                @pl.when(b + 2 < _NB)
                def _():
                    prep(b + 2, gbs[x2], k)
                    swait(b - 1, rws[x2], sss[x2])
                    gath(gbs[x2], rws[x2], gss[x2])

        slot(0, 0, True)
        slot(1, 1, False)
        slot(2, 2, False)

        def body(i, carry):
            b0 = 3 * i
            slot(b0, 0, False)
            slot(b0 + 1, 1, False)
            slot(b0 + 2, 2, False)
            return carry

        lax.fori_loop(1, _NB // 3, body, None)
        swait(_NB - 3, rws[0], sss[0])
        swait(_NB - 2, rws[1], sss[1])
        swait(_NB - 1, rws[2], sss[2])
        for j in range(_TAIL // _L):
            gb0[pl.ds(j * _L, _L)] = (
                srca[pl.ds(_NB * _B + j * _L, _L)] + k)
        pltpu.async_copy(
            xv_hbm.at[gb0.at[pl.ds(0, _TAIL)]],
            rw0.at[pl.ds(0, _TAIL)], gs0)
        pltpu.make_async_copy(
            xv_hbm.at[gb0.at[pl.ds(0, _TAIL)]],
            rw0.at[pl.ds(0, _TAIL)], gs0).wait()
        pltpu.sync_copy(
            rw0.at[pl.ds(0, _TAIL)],
            acc.at[sia.at[pl.ds(_NB * _B, _TAIL)]], add=True)

        plsc.subcore_barrier()
        pltpu.sync_copy(acc.at[pl.ds(base, _SPT)],
                        agg_out.at[pl.ds(base, _SPT), k])
        plsc.subcore_barrier()
        return carry

    lax.fori_loop(0, _CPC, chunk_pass, None)


def _make_sc_agg(with_count):
    out_types = [jax.ShapeDtypeStruct((_RN, _NCHUNK, _L), jnp.float32)]
    if with_count:
        out_types.append(jax.ShapeDtypeStruct((_NC, _RN, _L), jnp.float32))
    scratch = [
        pltpu.VMEM((_EPT,), jnp.int32),
        pltpu.VMEM((_EPT,), jnp.int32),
        pltpu.VMEM((_B,), jnp.int32),
        pltpu.VMEM((_B,), jnp.int32),
        pltpu.VMEM((_B,), jnp.int32),
        pltpu.VMEM((_B, _L), jnp.float32),
        pltpu.VMEM((_B, _L), jnp.float32),
        pltpu.VMEM((_B, _L), jnp.float32),
        pltpu.VMEM((_ZR, _L), jnp.float32),
        pltpu.VMEM_SHARED((_RN, _L), jnp.float32),
        pltpu.SemaphoreType.DMA,
        pltpu.SemaphoreType.DMA,
        pltpu.SemaphoreType.DMA,
        pltpu.SemaphoreType.DMA,
        pltpu.SemaphoreType.DMA,
        pltpu.SemaphoreType.DMA,
    ]
    return pl.kernel(
        functools.partial(_sc_agg_body, with_count),
        out_type=tuple(out_types) if with_count else out_types[0],
        mesh=_mesh,
        scratch_types=scratch,
        compiler_params=_sc_params,
    )


_sc_agg_cnt = _make_sc_agg(True)
_sc_agg = _make_sc_agg(False)


_BN = 400


def _tc_dense(x, agg, cnt, w_rel, w_root, b):
    agg3 = agg.reshape(_R, _N, _D)
    cnt4 = cnt.reshape(_NC, _R, _N, _L)
    b2 = b.reshape(1, _D)

    def body(x_ref, agg_ref, cnt_ref, wrel_ref, wroot_ref, b_ref, o_ref):
        xb = x_ref[...]
        accv = jnp.dot(xb, wroot_ref[...],
                       preferred_element_type=jnp.float32) + b_ref[...]
        cntv = cnt_ref[0] + cnt_ref[1]
        inv = 1.0 / jnp.maximum(cntv[:, :, :1], 1.0)
        for r in range(_R):
            m = agg_ref[r] * inv[r]
            accv = accv + jnp.dot(m, wrel_ref[r],
                                  preferred_element_type=jnp.float32)
        o_ref[...] = jnp.maximum(accv, 0.0)

    return pl.pallas_call(
        body,
        grid=(_N // _BN,),
        in_specs=[
            pl.BlockSpec((_BN, _D), lambda i: (i, 0)),
            pl.BlockSpec((_R, _BN, _D), lambda i: (0, i, 0)),
            pl.BlockSpec((_NC, _R, _BN, _L), lambda i: (0, 0, i, 0)),
            pl.BlockSpec((_R, _D, _D), lambda i: (0, 0, 0)),
            pl.BlockSpec((_D, _D), lambda i: (0, 0)),
            pl.BlockSpec((1, _D), lambda i: (0, 0)),
        ],
        out_specs=pl.BlockSpec((_BN, _D), lambda i: (i, 0)),
        out_shape=jax.ShapeDtypeStruct((_N, _D), jnp.float32),
    )(x, agg3, cnt4, w_rel, w_root, b2)


def kernel(x, edge_index, edge_type, edge_attr,
           W_rel1, W_root1, b1, W_rel2, W_root2, b2):
    del edge_attr
    src = edge_index[0]
    dst = edge_index[1]
    agg1, cnt = _sc_agg_cnt(x.reshape(_N * _NCHUNK, _L), src, dst, edge_type)
    h = _tc_dense(x, agg1, cnt, W_rel1, W_root1, b1)
    agg2 = _sc_agg(h.reshape(_N * _NCHUNK, _L), src, dst, edge_type)
    return _tc_dense(h, agg2, cnt, W_rel2, W_root2, b2)
